# baseline (speedup 1.0000x reference)
import jax
import jax.numpy as jnp
from jax.experimental import pallas as pl
from jax.experimental.pallas import tpu as pltpu

_B = 2
_N = 100000
_G = 50
_NEG = 0.3
_POS = 0.5
_ALPHA = 0.25
_BETA = 0.1
_LOSS_NORM = 100.0
_MOM = 0.9

_R = 8
_BLK = _R * 128
_NB = -(-_N // _BLK)
_NPAD = _NB * _BLK
_NROWS = _NPAD // 128


def _transform(ax0, ay0, ax1, ay1, gx0, gy0, gx1, gy1):
    bw = ax1 - ax0 + 1.0
    bh = ay1 - ay0 + 1.0
    bx = ax0 + 0.5 * bw
    by = ay0 + 0.5 * bh
    gw = gx1 - gx0 + 1.0
    gh = gy1 - gy0 + 1.0
    gx = gx0 + 0.5 * gw
    gy = gy0 + 0.5 * gh
    return (gx - bx) / bw, (gy - by) / bh, jnp.log(gw / bw), jnp.log(gh / bh)


def _obj(p, lab):
    pos = (lab == 1.0) * ((1.0 - p) * (1.0 - p)) * jnp.log(p) * _ALPHA
    neg = ((lab != 1.0) & (lab != -1.0)) * (p * p) * jnp.log(1.0 - p) * (1.0 - _ALPHA)
    return -(pos + neg)


def _sl1(pr, tg):
    s = None
    for a, b in zip(pr, tg):
        x = jnp.abs(a - b)
        t = jnp.where(x < _BETA, 0.5 * x * x / _BETA, x - 0.5 * _BETA)
        s = t if s is None else s + t
    return s


def _emd(p0, p1, r0, r1, lab0, lab1, t0, t1):
    v0 = (lab0 >= 0).astype(jnp.float32)
    f0 = (lab0 > 0).astype(jnp.float32)
    v1 = (lab1 >= 0).astype(jnp.float32)
    f1 = (lab1 > 0).astype(jnp.float32)
    l0 = (_obj(p0, lab0) * v0 + _sl1(r0, t0) * f0) + (_obj(p1, lab1) * v1 + _sl1(r1, t1) * f1)
    l1 = (_obj(p1, lab0) * v0 + _sl1(r1, t0) * f0) + (_obj(p0, lab1) * v1 + _sl1(r0, t1) * f1)
    return jnp.minimum(l0, l1)


def _finalize_labels(m, lab):
    lab = lab * (m >= _NEG)
    return jnp.where((m < _POS) & (m >= _NEG), -1.0, lab)


def _body_all(gt_ref, gtv_ref, gtlb_ref, a_ref, pc_ref, pr_ref, aw_ref, pcw_ref, prw_ref,
              loss_ref, npos_ref, vacc_ref, iacc_ref, out_ref,
              garg_v, tag_v, gs_ref, ga_lq, ga_own, gp_cls, gp_reg, sem):
    b = pl.program_id(0)
    i = pl.program_id(1)

    @pl.when(i < _NB)
    def _():
        first = i == 0
        ax0 = a_ref[0]
        ay0 = a_ref[1]
        ax1 = a_ref[2]
        ay1 = a_ref[3]
        aw = ax1 - ax0 + 1.0
        ah = ay1 - ay0 + 1.0
        area = aw * ah
        rows = jax.lax.broadcasted_iota(jnp.int32, (_R, 128), 0)
        lanes = jax.lax.broadcasted_iota(jnp.int32, (_R, 128), 1)
        gidx = (i * _R + rows) * 128 + lanes
        valid = gidx < _N
        rowpat = 7 - rows

        intmin = jnp.int32(-2**31)
        km0 = jnp.full((_R, 128), intmin)
        km1 = jnp.full((_R, 128), intmin)
        tbm = gtlb_ref[0]
        for g in range(_G):
            def _bc(r, g=g):
                return jnp.broadcast_to(tbm[r:r + 1, g:g + 1], (_R, 128))
            gx0 = _bc(0)
            gy0 = _bc(1)
            gx1 = _bc(2)
            gy1 = _bc(3)
            garea = _bc(5)
            ltx = jnp.maximum(ax0, gx0)
            lty = jnp.maximum(ay0, gy0)
            rbx = jnp.minimum(ax1, gx1)
            rby = jnp.minimum(ay1, gy1)
            w = jnp.maximum(rbx - ltx + 1.0, 0.0)
            h = jnp.maximum(rby - lty + 1.0, 0.0)
            inter = w * h
            iou = inter / ((area + garea) - inter)

            kb = jax.lax.bitcast_convert_type(iou, jnp.int32)
            keyg = jnp.bitwise_or(jnp.bitwise_and(kb, jnp.int32(-64)), jnp.int32(g))
            nk0 = jnp.maximum(km0, keyg)
            km1 = jnp.maximum(km1, jnp.minimum(km0, keyg))
            km0 = nk0

            iex = jnp.where(valid, iou, -1.0)
            kbits = jax.lax.bitcast_convert_type(iex, jnp.int32)
            key = jnp.bitwise_or(jnp.bitwise_and(kbits, jnp.int32(-8)), rowpat)
            kcol = jnp.max(key, axis=0, keepdims=True)
            cur = vacc_ref[0, g:g + 1, :]
            curb = iacc_ref[0, g:g + 1, :]
            better = jnp.logical_or(kcol > cur, first)
            vacc_ref[0, g:g + 1, :] = jnp.where(better, kcol, cur)
            iacc_ref[0, g:g + 1, :] = jnp.where(better, i, curb)

        i0 = jnp.bitwise_and(km0, 63)
        i1 = jnp.bitwise_and(km1, 63)
        m0 = jax.lax.bitcast_convert_type(jnp.bitwise_and(km0, jnp.int32(-64)),
                                          jnp.float32)
        m1 = jax.lax.bitcast_convert_type(jnp.bitwise_and(km1, jnp.int32(-64)),
                                          jnp.float32)
        tb = tbm

        def _tg(row, idx):
            t = jnp.broadcast_to(tb[row:row + 1, :], (_R, 128))
            return jnp.take_along_axis(t, idx, axis=1)

        la0 = _tg(4, i0)
        la1 = _tg(4, i1)
        b0 = [_tg(c, i0) for c in range(4)]
        b1 = [_tg(c, i1) for c in range(4)]
        lab0 = _finalize_labels(m0, la0)
        lab1 = _finalize_labels(m1, la1)
        t0 = _transform(ax0, ay0, ax1, ay1, b0[0], b0[1], b0[2], b0[3])
        t1 = _transform(ax0, ay0, ax1, ay1, b1[0], b1[1], b1[2], b1[3])
        p0 = jax.nn.sigmoid(pc_ref[0, 0])
        p1 = jax.nn.sigmoid(pc_ref[0, 1])
        r0 = [pr_ref[0, c] for c in range(4)]
        r1 = [pr_ref[0, c] for c in range(4, 8)]
        le = _emd(p0, p1, r0, r1, lab0, lab1, t0, t1)
        part = jnp.sum(jnp.where(valid, le, 0.0))
        npp = jnp.sum(jnp.where(jnp.logical_and(lab0 > 0, valid), 1.0, 0.0))
        start = jnp.logical_and(b == 0, i == 0)
        loss_ref[0, 0] = jnp.where(start, 0.0, loss_ref[0, 0]) + part
        npos_ref[0, 0] = jnp.where(start, 0.0, npos_ref[0, 0]) + npp

    @pl.when(i == _NB)
    def _():
        kacc = vacc_ref[0]
        bacc = iacc_ref[0]
        lane64 = jax.lax.broadcasted_iota(jnp.int32, (64, 128), 1)
        gcol = jax.lax.broadcasted_iota(jnp.int32, (64, 1), 0)
        km = jnp.max(kacc, axis=1, keepdims=True)
        row = 7 - jnp.bitwise_and(kacc, 7)
        full = (bacc * _R + row) * 128 + lane64
        cand = jnp.where(kacc == km, full, jnp.int32(2**30))
        idx = jnp.min(cand, axis=1, keepdims=True)
        garg_v[pl.ds(b * 64, 64), :] = jnp.where(gcol < _G, idx, 0)
        tag_v[pl.ds(b * 64, 64), :] = jnp.where(gcol < _G, idx + b * (1 << 20), -1)

    @pl.when(jnp.logical_and(b == _B - 1, i == _NB))
    def _():
        pltpu.make_async_copy(garg_v, gs_ref, sem).start()
        pltpu.make_async_copy(garg_v, gs_ref, sem).wait()

        def _cps(e):
            idx = gs_ref[e, 0]
            own = jax.lax.shift_right_logical(idx, 1)
            irow = jax.lax.shift_right_logical(idx, 7)
            orow = jax.lax.shift_right_logical(own, 7)
            eb = 0 if e < 64 else 1
            return (
                pltpu.make_async_copy(aw_ref.at[:, pl.ds(irow, 1), :],
                                      ga_lq.at[:, pl.ds(e, 1), :], sem),
                pltpu.make_async_copy(aw_ref.at[:, pl.ds(orow, 1), :],
                                      ga_own.at[:, pl.ds(e, 1), :], sem),
                pltpu.make_async_copy(pcw_ref.at[eb, :, pl.ds(orow, 1), :],
                                      gp_cls.at[:, pl.ds(e, 1), :], sem),
                pltpu.make_async_copy(prw_ref.at[eb, :, pl.ds(orow, 1), :],
                                      gp_reg.at[:, pl.ds(e, 1), :], sem),
            )

        for e in range(128):
            for cp in _cps(e):
                cp.start()
        for e in range(128):
            for cp in _cps(e):
                cp.wait()

        lanes = jax.lax.broadcasted_iota(jnp.int32, (128, 128), 1)
        gidx_col = garg_v[...]
        oidx_col = jax.lax.shift_right_logical(gidx_col, 1)
        o_lq = lanes == jnp.bitwise_and(gidx_col, 127)
        o_own = lanes == jnp.bitwise_and(oidx_col, 127)

        def _ext(mask, tab):
            return jnp.sum(jnp.where(mask, tab, 0.0), axis=1, keepdims=True)

        lqc = [_ext(o_lq, ga_lq[c]) for c in range(4)]
        ownc = [_ext(o_own, ga_own[c]) for c in range(4)]
        clsc = [_ext(o_own, gp_cls[c]) for c in range(2)]
        regc = [_ext(o_own, gp_reg[c]) for c in range(8)]

        lanes = jax.lax.broadcasted_iota(jnp.int32, (128, 128), 1)
        rowsq = jax.lax.broadcasted_iota(jnp.int32, (128, 128), 0)
        lane1 = jax.lax.broadcasted_iota(jnp.int32, (1, 128), 1)
        rowc = jax.lax.broadcasted_iota(jnp.int32, (128, 1), 0)
        g_r = jnp.bitwise_and(rowc, 63)
        row_b = jnp.right_shift(rowc, 6)
        lane_b = jnp.right_shift(lane1, 6)
        lane_g = jnp.bitwise_and(lane1, 63)

        eye = (rowsq == lanes).astype(jnp.float32)
        v = tag_v[...]
        dnum = (((0,), (0,)), ((), ()))
        glane_f = jax.lax.dot_general(v.astype(jnp.float32), eye, dnum,
                                      preferred_element_type=jnp.float32,
                                      precision=jax.lax.Precision.HIGHEST)
        glane = glane_f.astype(jnp.int32)
        vA = jnp.right_shift(v, 1)
        glaneA = jnp.right_shift(glane, 1)

        r50 = jax.lax.broadcasted_iota(jnp.int32, (_G, 128), 0)
        m50 = jax.lax.broadcasted_iota(jnp.int32, (_G, 128), 1)
        gtl = (jax.lax.dot_general(gtv_ref[0], (m50 == r50).astype(jnp.float32),
                                   dnum, preferred_element_type=jnp.float32,
                                   precision=jax.lax.Precision.HIGHEST)
               + jax.lax.dot_general(gtv_ref[1],
                                     (m50 == r50 + 64).astype(jnp.float32),
                                     dnum, preferred_element_type=jnp.float32,
                                     precision=jax.lax.Precision.HIGHEST))

        gx0 = gtl[0:1, :]
        gy0 = gtl[1:2, :]
        gx1 = gtl[2:3, :]
        gy1 = gtl[3:4, :]
        glab = gtl[4:5, :]

        ax0 = ownc[0]
        ay0 = ownc[1]
        ax1 = ownc[2]
        ay1 = ownc[3]
        aw = ax1 - ax0 + 1.0
        ah = ay1 - ay0 + 1.0
        area = aw * ah
        garea = (gx1 - gx0 + 1.0) * (gy1 - gy0 + 1.0)
        ltx = jnp.maximum(ax0, gx0)
        lty = jnp.maximum(ay0, gy0)
        rbx = jnp.minimum(ax1, gx1)
        rby = jnp.minimum(ay1, gy1)
        w = jnp.maximum(rbx - ltx + 1.0, 0.0)
        h = jnp.maximum(rby - lty + 1.0, 0.0)
        inter = w * h
        iou = inter / ((area + garea) - inter)

        samebatch = (lane_b == row_b) & (lane_g < _G)
        iex = jnp.where(samebatch, iou, -1.0)
        kb = jax.lax.bitcast_convert_type(iex, jnp.int32)
        keyl = jnp.bitwise_or(jnp.bitwise_and(kb, jnp.int32(-64)), lane_g)
        k0 = jnp.max(keyl, axis=1, keepdims=True)
        keyl2 = jnp.where(keyl == k0, jnp.int32(-2**31), keyl)
        k1 = jnp.max(keyl2, axis=1, keepdims=True)
        m0 = jax.lax.bitcast_convert_type(jnp.bitwise_and(k0, jnp.int32(-64)),
                                          jnp.float32)
        m1 = jax.lax.bitcast_convert_type(jnp.bitwise_and(k1, jnp.int32(-64)),
                                          jnp.float32)
        i0 = jnp.bitwise_and(k0, 63) + row_b * 64
        i1 = jnp.bitwise_and(k1, 63) + row_b * 64

        def _sel(tab, idx):
            return jnp.sum(jnp.where(lanes == idx, tab, 0.0), axis=1, keepdims=True)

        la0 = _sel(glab, i0)
        la1 = _sel(glab, i1)
        bs0 = [_sel(t, i0) for t in (gx0, gy0, gx1, gy1)]
        bs1 = [_sel(t, i1) for t in (gx0, gy0, gx1, gy1)]
        lab0 = _finalize_labels(m0, la0)
        lab1 = _finalize_labels(m1, la1)
        t0 = _transform(ax0, ay0, ax1, ay1, bs0[0], bs0[1], bs0[2], bs0[3])
        t1 = _transform(ax0, ay0, ax1, ay1, bs1[0], bs1[1], bs1[2], bs1[3])

        p0 = jax.nn.sigmoid(clsc[0])
        p1 = jax.nn.sigmoid(clsc[1])
        r0 = regc[:4]
        r1 = regc[4:]
        base = _emd(p0, p1, r0, r1, lab0, lab1, t0, t1)

        lqm = jnp.concatenate(lqc, axis=1)
        lqT = jax.lax.dot_general(lqm, eye, dnum,
                                  preferred_element_type=jnp.float32,
                                  precision=jax.lax.Precision.HIGHEST)
        lq = _transform(lqT[0:1, :], lqT[1:2, :], lqT[2:3, :], lqT[3:4, :],
                        gx0, gy0, gx1, gy1)

        labf = [None, None]
        tf = [None, None]
        for k in range(2):
            tgt = jnp.bitwise_or(jnp.bitwise_and(v, jnp.int32(-2)), jnp.int32(k))
            eq = glane == tgt
            win = jnp.max(jnp.where(eq, lanes, jnp.int32(-1)), axis=1, keepdims=True)
            has = win >= 0
            nl = _sel(glab, win)
            nt = [_sel(c, win) for c in lq]
            lb = lab0 if k == 0 else lab1
            tb = t0 if k == 0 else t1
            labf[k] = jnp.where(has, nl, lb)
            tf[k] = tuple(jnp.where(has, a, bq) for a, bq in zip(nt, tb))
        new = _emd(p0, p1, r0, r1, labf[0], labf[1], tf[0], tf[1])

        eqpA = (glaneA == vA) & (lanes < rowc)
        dup = jnp.max(jnp.where(eqpA, 1, 0), axis=1, keepdims=True)
        active = ((g_r < _G) & (dup == 0)).astype(jnp.float32)
        delta = jnp.sum((new - base) * active)
        dnp = jnp.sum((jnp.where(labf[0] > 0, 1.0, 0.0)
                       - jnp.where(lab0 > 0, 1.0, 0.0)) * active)
        total = loss_ref[0, 0] + delta
        npos = npos_ref[0, 0] + dnp
        norm = _MOM * _LOSS_NORM + (1.0 - _MOM) * jnp.maximum(npos, 1.0)
        out_ref[0, 0] = total / norm


def _run_all(gt, gtlb, a4, pc, pr, *, interpret=False):
    f32 = jnp.float32
    i32 = jnp.int32
    nbm1 = _NB - 1
    return pl.pallas_call(
        _body_all,
        grid=(_B, _NB + 1),
        in_specs=[
            pl.BlockSpec((_B, _G, 5), lambda b, i: (0, 0, 0), memory_space=pltpu.SMEM),
            pl.BlockSpec((_B, _G, 5), lambda b, i: (0, 0, 0)),
            pl.BlockSpec((1, 8, 128), lambda b, i: (b, 0, 0)),
            pl.BlockSpec((4, _R, 128), lambda b, i: (0, jnp.minimum(i, nbm1), 0)),
            pl.BlockSpec((1, 2, _R, 128), lambda b, i: (b, 0, jnp.minimum(i, nbm1), 0)),
            pl.BlockSpec((1, 8, _R, 128), lambda b, i: (b, 0, jnp.minimum(i, nbm1), 0)),
            pl.BlockSpec(memory_space=pl.ANY),
            pl.BlockSpec(memory_space=pl.ANY),
            pl.BlockSpec(memory_space=pl.ANY),
        ],
        out_specs=[
            pl.BlockSpec((1, 1), lambda b, i: (0, 0), memory_space=pltpu.SMEM),
            pl.BlockSpec((1, 1), lambda b, i: (0, 0), memory_space=pltpu.SMEM),
            pl.BlockSpec((1, 64, 128), lambda b, i: (b, 0, 0)),
            pl.BlockSpec((1, 64, 128), lambda b, i: (b, 0, 0)),
            pl.BlockSpec((1, 1), lambda b, i: (0, 0), memory_space=pltpu.SMEM),
        ],
        out_shape=[
            jax.ShapeDtypeStruct((1, 1), f32),
            jax.ShapeDtypeStruct((1, 1), f32),
            jax.ShapeDtypeStruct((_B, 64, 128), i32),
            jax.ShapeDtypeStruct((_B, 64, 128), i32),
            jax.ShapeDtypeStruct((1, 1), f32),
        ],
        scratch_shapes=[
            pltpu.VMEM((_B * 64, 1), i32),
            pltpu.VMEM((_B * 64, 1), i32),
            pltpu.SMEM((_B * 64, 1), i32),
            pltpu.VMEM((4, 128, 128), f32),
            pltpu.VMEM((4, 128, 128), f32),
            pltpu.VMEM((2, 128, 128), f32),
            pltpu.VMEM((8, 128, 128), f32),
            pltpu.SemaphoreType.DMA,
        ],
        interpret=interpret,
    )(gt, gt, gtlb, a4, pc, pr, a4, pc, pr)


def kernel(pred_cls, pred_reg, anchors, gt_boxes, im_info):
    f32 = jnp.float32
    pad = _NPAD - _N
    at = anchors.T
    padbox = jnp.tile(jnp.array([[0.0], [0.0], [15.0], [15.0]], f32), (1, pad))
    a4 = jnp.concatenate([at, padbox], axis=1).reshape(4, _NROWS, 128)
    pc = jnp.pad(pred_cls, ((0, 0), (0, pad), (0, 0))).transpose(0, 2, 1)
    pc = pc.reshape(_B, 2, _NROWS, 128)
    pr = jnp.pad(pred_reg, ((0, 0), (0, pad), (0, 0))).transpose(0, 2, 1)
    pr = pr.reshape(_B, 8, _NROWS, 128)
    garea_x = ((gt_boxes[..., 2] - gt_boxes[..., 0] + 1.0)
               * (gt_boxes[..., 3] - gt_boxes[..., 1] + 1.0))
    gtlb = jnp.concatenate([jnp.transpose(gt_boxes, (0, 2, 1)),
                            garea_x[:, None, :]], axis=1)
    gtlb = jnp.pad(gtlb, ((0, 0), (0, 2), (0, 78)))

    outs = _run_all(gt_boxes, gtlb, a4, pc, pr)
    return outs[4][0, 0]


# speedup vs baseline: 1.5831x; 1.5831x over previous
import jax
import jax.numpy as jnp
from jax.experimental import pallas as pl
from jax.experimental.pallas import tpu as pltpu

_B = 2
_N = 100000
_G = 50
_NEG = 0.3
_POS = 0.5
_ALPHA = 0.25
_BETA = 0.1
_LOSS_NORM = 100.0
_MOM = 0.9

_R = 16
_BLK = _R * 128
_NB = -(-_N // _BLK)
_NPAD = _NB * _BLK
_NROWS = _NPAD // 128


def _transform(ax0, ay0, ax1, ay1, gx0, gy0, gx1, gy1):
    bw = ax1 - ax0 + 1.0
    bh = ay1 - ay0 + 1.0
    bx = ax0 + 0.5 * bw
    by = ay0 + 0.5 * bh
    gw = gx1 - gx0 + 1.0
    gh = gy1 - gy0 + 1.0
    gx = gx0 + 0.5 * gw
    gy = gy0 + 0.5 * gh
    return (gx - bx) / bw, (gy - by) / bh, jnp.log(gw / bw), jnp.log(gh / bh)


def _obj(p, lab):
    pos = (lab == 1.0) * ((1.0 - p) * (1.0 - p)) * jnp.log(p) * _ALPHA
    neg = ((lab != 1.0) & (lab != -1.0)) * (p * p) * jnp.log(1.0 - p) * (1.0 - _ALPHA)
    return -(pos + neg)


def _sl1(pr, tg):
    s = None
    for a, b in zip(pr, tg):
        x = jnp.abs(a - b)
        t = jnp.where(x < _BETA, 0.5 * x * x / _BETA, x - 0.5 * _BETA)
        s = t if s is None else s + t
    return s


def _emd(p0, p1, r0, r1, lab0, lab1, t0, t1):
    v0 = (lab0 >= 0).astype(jnp.float32)
    f0 = (lab0 > 0).astype(jnp.float32)
    v1 = (lab1 >= 0).astype(jnp.float32)
    f1 = (lab1 > 0).astype(jnp.float32)
    l0 = (_obj(p0, lab0) * v0 + _sl1(r0, t0) * f0) + (_obj(p1, lab1) * v1 + _sl1(r1, t1) * f1)
    l1 = (_obj(p1, lab0) * v0 + _sl1(r1, t0) * f0) + (_obj(p0, lab1) * v1 + _sl1(r0, t1) * f1)
    return jnp.minimum(l0, l1)


def _finalize_labels(m, lab):
    lab = lab * (m >= _NEG)
    return jnp.where((m < _POS) & (m >= _NEG), -1.0, lab)


def _body_all(gt_ref, gtv_ref, gtlb_ref, a_ref, pc_ref, pr_ref, aw_ref, pcw_ref, prw_ref,
              loss_ref, npos_ref, vacc_ref, iacc_ref, out_ref,
              garg_v, tag_v, gs_ref, ga_lq, ga_own, gp_cls, gp_reg, sem):
    b = pl.program_id(0)
    i = pl.program_id(1)

    @pl.when(i < _NB)
    def _():
        first = i == 0
        ax0 = a_ref[0]
        ay0 = a_ref[1]
        ax1 = a_ref[2]
        ay1 = a_ref[3]
        aw = ax1 - ax0 + 1.0
        ah = ay1 - ay0 + 1.0
        area = aw * ah
        rows = jax.lax.broadcasted_iota(jnp.int32, (_R, 128), 0)
        lanes = jax.lax.broadcasted_iota(jnp.int32, (_R, 128), 1)
        gidx = (i * _R + rows) * 128 + lanes
        valid = gidx < _N
        rowpat = (_R - 1) - rows

        intmin = jnp.int32(-2**31)
        km0 = jnp.full((_R, 128), intmin)
        km1 = jnp.full((_R, 128), intmin)
        for g in range(_G):
            gx0 = gt_ref[b, g, 0]
            gy0 = gt_ref[b, g, 1]
            gx1 = gt_ref[b, g, 2]
            gy1 = gt_ref[b, g, 3]
            garea = (gx1 - gx0 + 1.0) * (gy1 - gy0 + 1.0)
            ltx = jnp.maximum(ax0, gx0)
            lty = jnp.maximum(ay0, gy0)
            rbx = jnp.minimum(ax1, gx1)
            rby = jnp.minimum(ay1, gy1)
            w = jnp.maximum(rbx - ltx + 1.0, 0.0)
            h = jnp.maximum(rby - lty + 1.0, 0.0)
            inter = w * h
            iou = inter / ((area + garea) - inter)

            kb = jax.lax.bitcast_convert_type(iou, jnp.int32)
            keyg = jnp.bitwise_or(jnp.bitwise_and(kb, jnp.int32(-64)), jnp.int32(g))
            nk0 = jnp.maximum(km0, keyg)
            km1 = jnp.maximum(km1, jnp.minimum(km0, keyg))
            km0 = nk0

            iex = jnp.where(valid, iou, -1.0)
            kbits = jax.lax.bitcast_convert_type(iex, jnp.int32)
            key = jnp.bitwise_or(jnp.bitwise_and(kbits, jnp.int32(-_R)), rowpat)
            kcol = jnp.max(key, axis=0, keepdims=True)
            cur = vacc_ref[0, g:g + 1, :]
            curb = iacc_ref[0, g:g + 1, :]
            better = jnp.logical_or(kcol > cur, first)
            vacc_ref[0, g:g + 1, :] = jnp.where(better, kcol, cur)
            iacc_ref[0, g:g + 1, :] = jnp.where(better, i, curb)

        i0 = jnp.bitwise_and(km0, 63)
        i1 = jnp.bitwise_and(km1, 63)
        m0 = jax.lax.bitcast_convert_type(jnp.bitwise_and(km0, jnp.int32(-64)),
                                          jnp.float32)
        m1 = jax.lax.bitcast_convert_type(jnp.bitwise_and(km1, jnp.int32(-64)),
                                          jnp.float32)
        tb = gtlb_ref[0]

        def _tg(row, idx):
            t = jnp.broadcast_to(tb[row:row + 1, :], (_R, 128))
            return jnp.take_along_axis(t, idx, axis=1)

        la0 = _tg(4, i0)
        la1 = _tg(4, i1)
        b0 = [_tg(c, i0) for c in range(4)]
        b1 = [_tg(c, i1) for c in range(4)]
        lab0 = _finalize_labels(m0, la0)
        lab1 = _finalize_labels(m1, la1)
        t0 = _transform(ax0, ay0, ax1, ay1, b0[0], b0[1], b0[2], b0[3])
        t1 = _transform(ax0, ay0, ax1, ay1, b1[0], b1[1], b1[2], b1[3])
        p0 = jax.nn.sigmoid(pc_ref[0, 0])
        p1 = jax.nn.sigmoid(pc_ref[0, 1])
        r0 = [pr_ref[0, c] for c in range(4)]
        r1 = [pr_ref[0, c] for c in range(4, 8)]
        le = _emd(p0, p1, r0, r1, lab0, lab1, t0, t1)
        part = jnp.sum(jnp.where(valid, le, 0.0))
        npp = jnp.sum(jnp.where(jnp.logical_and(lab0 > 0, valid), 1.0, 0.0))
        start = jnp.logical_and(b == 0, i == 0)
        loss_ref[0, 0] = jnp.where(start, 0.0, loss_ref[0, 0]) + part
        npos_ref[0, 0] = jnp.where(start, 0.0, npos_ref[0, 0]) + npp

    @pl.when(i == _NB)
    def _():
        kacc = vacc_ref[0]
        bacc = iacc_ref[0]
        lane64 = jax.lax.broadcasted_iota(jnp.int32, (64, 128), 1)
        gcol = jax.lax.broadcasted_iota(jnp.int32, (64, 1), 0)
        km = jnp.max(kacc, axis=1, keepdims=True)
        row = (_R - 1) - jnp.bitwise_and(kacc, _R - 1)
        full = (bacc * _R + row) * 128 + lane64
        cand = jnp.where(kacc == km, full, jnp.int32(2**30))
        idx = jnp.min(cand, axis=1, keepdims=True)
        garg_v[pl.ds(b * 64, 64), :] = jnp.where(gcol < _G, idx, 0)
        tag_v[pl.ds(b * 64, 64), :] = jnp.where(gcol < _G, idx + b * (1 << 20), -1)

    @pl.when(jnp.logical_and(b == _B - 1, i == _NB))
    def _():
        pltpu.make_async_copy(garg_v, gs_ref, sem).start()
        pltpu.make_async_copy(garg_v, gs_ref, sem).wait()

        def _cps(e):
            idx = gs_ref[e, 0]
            own = jax.lax.shift_right_logical(idx, 1)
            irow = jax.lax.shift_right_logical(idx, 7)
            orow = jax.lax.shift_right_logical(own, 7)
            eb = 0 if e < 64 else 1
            return (
                pltpu.make_async_copy(aw_ref.at[:, pl.ds(irow, 1), :],
                                      ga_lq.at[:, pl.ds(e, 1), :], sem),
                pltpu.make_async_copy(aw_ref.at[:, pl.ds(orow, 1), :],
                                      ga_own.at[:, pl.ds(e, 1), :], sem),
                pltpu.make_async_copy(pcw_ref.at[eb, :, pl.ds(orow, 1), :],
                                      gp_cls.at[:, pl.ds(e, 1), :], sem),
                pltpu.make_async_copy(prw_ref.at[eb, :, pl.ds(orow, 1), :],
                                      gp_reg.at[:, pl.ds(e, 1), :], sem),
            )

        for e in range(128):
            for cp in _cps(e):
                cp.start()
        for e in range(128):
            for cp in _cps(e):
                cp.wait()

        lanes = jax.lax.broadcasted_iota(jnp.int32, (128, 128), 1)
        gidx_col = garg_v[...]
        oidx_col = jax.lax.shift_right_logical(gidx_col, 1)
        o_lq = lanes == jnp.bitwise_and(gidx_col, 127)
        o_own = lanes == jnp.bitwise_and(oidx_col, 127)

        def _ext(mask, tab):
            return jnp.sum(jnp.where(mask, tab, 0.0), axis=1, keepdims=True)

        lqc = [_ext(o_lq, ga_lq[c]) for c in range(4)]
        ownc = [_ext(o_own, ga_own[c]) for c in range(4)]
        clsc = [_ext(o_own, gp_cls[c]) for c in range(2)]
        regc = [_ext(o_own, gp_reg[c]) for c in range(8)]

        lanes = jax.lax.broadcasted_iota(jnp.int32, (128, 128), 1)
        rowsq = jax.lax.broadcasted_iota(jnp.int32, (128, 128), 0)
        lane1 = jax.lax.broadcasted_iota(jnp.int32, (1, 128), 1)
        rowc = jax.lax.broadcasted_iota(jnp.int32, (128, 1), 0)
        g_r = jnp.bitwise_and(rowc, 63)
        row_b = jnp.right_shift(rowc, 6)
        lane_b = jnp.right_shift(lane1, 6)
        lane_g = jnp.bitwise_and(lane1, 63)

        eye = (rowsq == lanes).astype(jnp.float32)
        v = tag_v[...]
        dnum = (((0,), (0,)), ((), ()))
        glane_f = jax.lax.dot_general(v.astype(jnp.float32), eye, dnum,
                                      preferred_element_type=jnp.float32,
                                      precision=jax.lax.Precision.HIGHEST)
        glane = glane_f.astype(jnp.int32)
        vA = jnp.right_shift(v, 1)
        glaneA = jnp.right_shift(glane, 1)

        r50 = jax.lax.broadcasted_iota(jnp.int32, (_G, 128), 0)
        m50 = jax.lax.broadcasted_iota(jnp.int32, (_G, 128), 1)
        gtl = (jax.lax.dot_general(gtv_ref[0], (m50 == r50).astype(jnp.float32),
                                   dnum, preferred_element_type=jnp.float32,
                                   precision=jax.lax.Precision.HIGHEST)
               + jax.lax.dot_general(gtv_ref[1],
                                     (m50 == r50 + 64).astype(jnp.float32),
                                     dnum, preferred_element_type=jnp.float32,
                                     precision=jax.lax.Precision.HIGHEST))

        gx0 = gtl[0:1, :]
        gy0 = gtl[1:2, :]
        gx1 = gtl[2:3, :]
        gy1 = gtl[3:4, :]
        glab = gtl[4:5, :]

        ax0 = ownc[0]
        ay0 = ownc[1]
        ax1 = ownc[2]
        ay1 = ownc[3]
        aw = ax1 - ax0 + 1.0
        ah = ay1 - ay0 + 1.0
        area = aw * ah
        garea = (gx1 - gx0 + 1.0) * (gy1 - gy0 + 1.0)
        ltx = jnp.maximum(ax0, gx0)
        lty = jnp.maximum(ay0, gy0)
        rbx = jnp.minimum(ax1, gx1)
        rby = jnp.minimum(ay1, gy1)
        w = jnp.maximum(rbx - ltx + 1.0, 0.0)
        h = jnp.maximum(rby - lty + 1.0, 0.0)
        inter = w * h
        iou = inter / ((area + garea) - inter)

        samebatch = (lane_b == row_b) & (lane_g < _G)
        iex = jnp.where(samebatch, iou, -1.0)
        kb = jax.lax.bitcast_convert_type(iex, jnp.int32)
        keyl = jnp.bitwise_or(jnp.bitwise_and(kb, jnp.int32(-64)), lane_g)
        k0 = jnp.max(keyl, axis=1, keepdims=True)
        keyl2 = jnp.where(keyl == k0, jnp.int32(-2**31), keyl)
        k1 = jnp.max(keyl2, axis=1, keepdims=True)
        m0 = jax.lax.bitcast_convert_type(jnp.bitwise_and(k0, jnp.int32(-64)),
                                          jnp.float32)
        m1 = jax.lax.bitcast_convert_type(jnp.bitwise_and(k1, jnp.int32(-64)),
                                          jnp.float32)
        i0 = jnp.bitwise_and(k0, 63) + row_b * 64
        i1 = jnp.bitwise_and(k1, 63) + row_b * 64

        def _sel(tab, idx):
            return jnp.sum(jnp.where(lanes == idx, tab, 0.0), axis=1, keepdims=True)

        la0 = _sel(glab, i0)
        la1 = _sel(glab, i1)
        bs0 = [_sel(t, i0) for t in (gx0, gy0, gx1, gy1)]
        bs1 = [_sel(t, i1) for t in (gx0, gy0, gx1, gy1)]
        lab0 = _finalize_labels(m0, la0)
        lab1 = _finalize_labels(m1, la1)
        t0 = _transform(ax0, ay0, ax1, ay1, bs0[0], bs0[1], bs0[2], bs0[3])
        t1 = _transform(ax0, ay0, ax1, ay1, bs1[0], bs1[1], bs1[2], bs1[3])

        p0 = jax.nn.sigmoid(clsc[0])
        p1 = jax.nn.sigmoid(clsc[1])
        r0 = regc[:4]
        r1 = regc[4:]
        base = _emd(p0, p1, r0, r1, lab0, lab1, t0, t1)

        lqm = jnp.concatenate(lqc, axis=1)
        lqT = jax.lax.dot_general(lqm, eye, dnum,
                                  preferred_element_type=jnp.float32,
                                  precision=jax.lax.Precision.HIGHEST)
        lq = _transform(lqT[0:1, :], lqT[1:2, :], lqT[2:3, :], lqT[3:4, :],
                        gx0, gy0, gx1, gy1)

        labf = [None, None]
        tf = [None, None]
        for k in range(2):
            tgt = jnp.bitwise_or(jnp.bitwise_and(v, jnp.int32(-2)), jnp.int32(k))
            eq = glane == tgt
            win = jnp.max(jnp.where(eq, lanes, jnp.int32(-1)), axis=1, keepdims=True)
            has = win >= 0
            nl = _sel(glab, win)
            nt = [_sel(c, win) for c in lq]
            lb = lab0 if k == 0 else lab1
            tb = t0 if k == 0 else t1
            labf[k] = jnp.where(has, nl, lb)
            tf[k] = tuple(jnp.where(has, a, bq) for a, bq in zip(nt, tb))
        new = _emd(p0, p1, r0, r1, labf[0], labf[1], tf[0], tf[1])

        eqpA = (glaneA == vA) & (lanes < rowc)
        dup = jnp.max(jnp.where(eqpA, 1, 0), axis=1, keepdims=True)
        active = ((g_r < _G) & (dup == 0)).astype(jnp.float32)
        delta = jnp.sum((new - base) * active)
        dnp = jnp.sum((jnp.where(labf[0] > 0, 1.0, 0.0)
                       - jnp.where(lab0 > 0, 1.0, 0.0)) * active)
        total = loss_ref[0, 0] + delta
        npos = npos_ref[0, 0] + dnp
        norm = _MOM * _LOSS_NORM + (1.0 - _MOM) * jnp.maximum(npos, 1.0)
        out_ref[0, 0] = total / norm


def _run_all(gt, gtlb, a4, pc, pr, *, interpret=False):
    f32 = jnp.float32
    i32 = jnp.int32
    nbm1 = _NB - 1
    return pl.pallas_call(
        _body_all,
        grid=(_B, _NB + 1),
        in_specs=[
            pl.BlockSpec((_B, _G, 5), lambda b, i: (0, 0, 0), memory_space=pltpu.SMEM),
            pl.BlockSpec((_B, _G, 5), lambda b, i: (0, 0, 0)),
            pl.BlockSpec((1, 8, 128), lambda b, i: (b, 0, 0)),
            pl.BlockSpec((4, _R, 128), lambda b, i: (0, jnp.minimum(i, nbm1), 0)),
            pl.BlockSpec((1, 2, _R, 128), lambda b, i: (b, 0, jnp.minimum(i, nbm1), 0)),
            pl.BlockSpec((1, 8, _R, 128), lambda b, i: (b, 0, jnp.minimum(i, nbm1), 0)),
            pl.BlockSpec(memory_space=pl.ANY),
            pl.BlockSpec(memory_space=pl.ANY),
            pl.BlockSpec(memory_space=pl.ANY),
        ],
        out_specs=[
            pl.BlockSpec((1, 1), lambda b, i: (0, 0), memory_space=pltpu.SMEM),
            pl.BlockSpec((1, 1), lambda b, i: (0, 0), memory_space=pltpu.SMEM),
            pl.BlockSpec((1, 64, 128), lambda b, i: (b, 0, 0)),
            pl.BlockSpec((1, 64, 128), lambda b, i: (b, 0, 0)),
            pl.BlockSpec((1, 1), lambda b, i: (0, 0), memory_space=pltpu.SMEM),
        ],
        out_shape=[
            jax.ShapeDtypeStruct((1, 1), f32),
            jax.ShapeDtypeStruct((1, 1), f32),
            jax.ShapeDtypeStruct((_B, 64, 128), i32),
            jax.ShapeDtypeStruct((_B, 64, 128), i32),
            jax.ShapeDtypeStruct((1, 1), f32),
        ],
        scratch_shapes=[
            pltpu.VMEM((_B * 64, 1), i32),
            pltpu.VMEM((_B * 64, 1), i32),
            pltpu.SMEM((_B * 64, 1), i32),
            pltpu.VMEM((4, 128, 128), f32),
            pltpu.VMEM((4, 128, 128), f32),
            pltpu.VMEM((2, 128, 128), f32),
            pltpu.VMEM((8, 128, 128), f32),
            pltpu.SemaphoreType.DMA,
        ],
        interpret=interpret,
    )(gt, gt, gtlb, a4, pc, pr, a4, pc, pr)


def kernel(pred_cls, pred_reg, anchors, gt_boxes, im_info):
    f32 = jnp.float32
    pad = _NPAD - _N
    at = anchors.T
    padbox = jnp.tile(jnp.array([[0.0], [0.0], [15.0], [15.0]], f32), (1, pad))
    a4 = jnp.concatenate([at, padbox], axis=1).reshape(4, _NROWS, 128)
    pc = jnp.pad(pred_cls, ((0, 0), (0, pad), (0, 0))).transpose(0, 2, 1)
    pc = pc.reshape(_B, 2, _NROWS, 128)
    pr = jnp.pad(pred_reg, ((0, 0), (0, pad), (0, 0))).transpose(0, 2, 1)
    pr = pr.reshape(_B, 8, _NROWS, 128)
    garea_x = ((gt_boxes[..., 2] - gt_boxes[..., 0] + 1.0)
               * (gt_boxes[..., 3] - gt_boxes[..., 1] + 1.0))
    gtlb = jnp.concatenate([jnp.transpose(gt_boxes, (0, 2, 1)),
                            garea_x[:, None, :]], axis=1)
    gtlb = jnp.pad(gtlb, ((0, 0), (0, 2), (0, 78)))

    outs = _run_all(gt_boxes, gtlb, a4, pc, pr)
    return outs[4][0, 0]
